# baseline (speedup 1.0000x reference)
import jax
import jax.numpy as jnp
from jax import lax
from jax.experimental import pallas as pl
from jax.experimental.pallas import tpu as pltpu
from jax.experimental.pallas import tpu_sc as plsc

SEQ = 200
D = 64
LANES = 16
NC, NS = 2, 16
NW = NC * NS
BB = 128
NG = D // 8


def _body(tok_hbm, word_hbm, pos_hbm, out_hbm,
          tokblk, idx_t, pos_v, gb0, gb1, ch0, ch1, sg0, sg1, sw0, sw1):
    gb = [gb0, gb1]
    ch = [ch0, ch1]
    sg = [sg0, sg1]
    sw = [sw0, sw1]
    wid = lax.axis_index("s") * NC + lax.axis_index("c")

    pltpu.sync_copy(tok_hbm.at[pl.ds(wid * BB, BB)], tokblk)
    pltpu.sync_copy(pos_hbm, pos_v)

    lane = lax.iota(jnp.int32, LANES)

    def tpose(b, c):
        for k in range(13):
            off = k * LANES if k < 12 else SEQ - LANES
            val = tokblk[b, pl.ds(off, LANES)]
            rowi = off + lane
            coli = jnp.full((LANES,), b, jnp.int32)
            if k == 12:
                plsc.store_scatter(idx_t, [rowi, coli], val,
                                   mask=rowi >= k * LANES)
            else:
                plsc.store_scatter(idx_t, [rowi, coli], val)
        return c

    lax.fori_loop(0, BB, tpose, 0)

    dummy = word_hbm.at[pl.ds(0, BB)]

    def issue_gather(s, par):
        pltpu.async_copy(word_hbm.at[idx_t.at[s]], gb[par], sg[par])

    issue_gather(0, 0)

    base = [(2 * k + lane // 8) * 1024 + (lane % 8) * 128 for k in range(4)]

    def step(s, par):
        @pl.when(s + 1 < SEQ)
        def _():
            issue_gather(s + 1, 1 - par)

        pltpu.make_async_copy(dummy, gb[par], sg[par]).wait()

        @pl.when(s >= 2)
        def _():
            for g in range(NG):
                pltpu.make_async_copy(ch[par].at[pl.ds(g * 1024, 1024)],
                                      out_hbm.at[0, g, 0], sw[par]).wait()

        pos_r = [pos_v[s, pl.ds(k * LANES, LANES)] for k in range(4)]

        def add_tp(b2, c):
            for u in range(2):
                b = 2 * b2 + u
                bvec = jnp.full((LANES,), b, jnp.int32)
                for k in range(4):
                    val = gb[par][b, pl.ds(k * LANES, LANES)] + pos_r[k]
                    plsc.store_scatter(ch[par], [base[k] + bvec], val)
            return c

        lax.fori_loop(0, BB // 2, add_tp, 0)
        for g in range(NG):
            pltpu.async_copy(ch[par].at[pl.ds(g * 1024, 1024)],
                             out_hbm.at[s, g, wid], sw[par])

    def pair(p, carry):
        step(2 * p, 0)
        step(2 * p + 1, 1)
        return carry

    lax.fori_loop(0, SEQ // 2, pair, 0)
    for par in range(2):
        for g in range(NG):
            pltpu.make_async_copy(ch[par].at[pl.ds(g * 1024, 1024)],
                                  out_hbm.at[0, g, 0], sw[par]).wait()


def kernel(tokens, word_table, pos_table):
    b, s = tokens.shape
    d = word_table.shape[1]
    tok = tokens.astype(jnp.int32)

    mesh = plsc.VectorSubcoreMesh(core_axis_name="c", subcore_axis_name="s",
                                  num_cores=NC, num_subcores=NS)
    run = pl.kernel(
        _body,
        out_type=jax.ShapeDtypeStruct((s, NG, b // BB, 8 * BB), jnp.float32),
        mesh=mesh,
        scratch_types=(
            [pltpu.VMEM((BB, s), jnp.int32),
             pltpu.VMEM((s, BB), jnp.int32),
             pltpu.VMEM((SEQ, d), jnp.float32),
             pltpu.VMEM((BB, d), jnp.float32),
             pltpu.VMEM((BB, d), jnp.float32),
             pltpu.VMEM((8 * BB * NG,), jnp.float32),
             pltpu.VMEM((8 * BB * NG,), jnp.float32)]
            + [pltpu.SemaphoreType.DMA for _ in range(4)]
        ),
        compiler_params=pltpu.CompilerParams(use_tc_tiling_on_sc=False,
                                             needs_layout_passes=False),
    )
    out = run(tok, word_table, pos_table)
    out = out.reshape(s, NG, b // BB, 8, BB)
    out = out.transpose(2, 4, 0, 1, 3).reshape(b, s, d)
    return out


# speedup vs baseline: 1.5281x; 1.5281x over previous
import jax
import jax.numpy as jnp
from jax import lax
from jax.experimental import pallas as pl
from jax.experimental.pallas import tpu as pltpu
from jax.experimental.pallas import tpu_sc as plsc

SEQ = 200
D = 64
LANES = 16
NC, NS = 2, 16
NW = NC * NS
BB = 128
NG = D // 8


def _body(tok_hbm, word_hbm, pos_hbm, out_hbm,
          tokblk, idx_t, pos_v, gb0, gb1, ch0, ch1, sg0, sg1, sw0, sw1):
    gb = [gb0, gb1]
    ch = [ch0, ch1]
    sg = [sg0, sg1]
    sw = [sw0, sw1]
    wid = lax.axis_index("s") * NC + lax.axis_index("c")

    pltpu.sync_copy(tok_hbm.at[pl.ds(wid * BB, BB)], tokblk)
    pltpu.sync_copy(pos_hbm, pos_v)

    lane = lax.iota(jnp.int32, LANES)

    def tpose(b, c):
        for k in range(13):
            off = k * LANES if k < 12 else SEQ - LANES
            val = tokblk[b, pl.ds(off, LANES)]
            rowi = off + lane
            coli = jnp.full((LANES,), b, jnp.int32)
            if k == 12:
                plsc.store_scatter(idx_t, [rowi, coli], val,
                                   mask=rowi >= k * LANES)
            else:
                plsc.store_scatter(idx_t, [rowi, coli], val)
        return c

    lax.fori_loop(0, BB, tpose, 0)

    dummy = word_hbm.at[pl.ds(0, BB)]

    def issue_gather(s, par):
        pltpu.async_copy(word_hbm.at[idx_t.at[s]], gb[par], sg[par])

    issue_gather(0, 0)

    gvec = [2 * k + lane // 8 for k in range(4)]
    cvec = lane % 8

    def step(s, par):
        @pl.when(s + 1 < SEQ)
        def _():
            issue_gather(s + 1, 1 - par)

        pltpu.make_async_copy(dummy, gb[par], sg[par]).wait()

        @pl.when(s >= 2)
        def _():
            for g in range(NG):
                pltpu.make_async_copy(ch[par].at[g, :, pl.ds(0, BB)],
                                      out_hbm.at[0, g, 0], sw[par]).wait()

        pos_r = [pos_v[s, pl.ds(k * LANES, LANES)] for k in range(4)]

        def add_tp(b2, c):
            for u in range(2):
                b = 2 * b2 + u
                bvec = jnp.full((LANES,), b, jnp.int32)
                for k in range(4):
                    val = gb[par][b, pl.ds(k * LANES, LANES)] + pos_r[k]
                    plsc.store_scatter(ch[par], [gvec[k], cvec, bvec], val)
            return c

        lax.fori_loop(0, BB // 2, add_tp, 0)
        for g in range(NG):
            pltpu.async_copy(ch[par].at[g, :, pl.ds(0, BB)],
                             out_hbm.at[s, g, wid], sw[par])

    def pair(p, carry):
        step(2 * p, 0)
        step(2 * p + 1, 1)
        return carry

    lax.fori_loop(0, SEQ // 2, pair, 0)
    for par in range(2):
        for g in range(NG):
            pltpu.make_async_copy(ch[par].at[g, :, pl.ds(0, BB)],
                                  out_hbm.at[0, g, 0], sw[par]).wait()


def kernel(tokens, word_table, pos_table):
    b, s = tokens.shape
    d = word_table.shape[1]
    tok = tokens.astype(jnp.int32)

    mesh = plsc.VectorSubcoreMesh(core_axis_name="c", subcore_axis_name="s",
                                  num_cores=NC, num_subcores=NS)
    run = pl.kernel(
        _body,
        out_type=jax.ShapeDtypeStruct((s, NG, b // BB, 8, BB), jnp.float32),
        mesh=mesh,
        scratch_types=(
            [pltpu.VMEM((BB, s), jnp.int32),
             pltpu.VMEM((s, BB), jnp.int32),
             pltpu.VMEM((SEQ, d), jnp.float32),
             pltpu.VMEM((BB, d), jnp.float32),
             pltpu.VMEM((BB, d), jnp.float32),
             pltpu.VMEM((NG, 8, BB + 1), jnp.float32),
             pltpu.VMEM((NG, 8, BB + 1), jnp.float32)]
            + [pltpu.SemaphoreType.DMA for _ in range(4)]
        ),
        compiler_params=pltpu.CompilerParams(use_tc_tiling_on_sc=False,
                                             needs_layout_passes=False),
    )
    out = run(tok, word_table, pos_table)
    return out.transpose(2, 4, 0, 1, 3).reshape(b, s, d)


# speedup vs baseline: 1.5281x; 1.0001x over previous
import jax
import jax.numpy as jnp
from jax import lax
from jax.experimental import pallas as pl
from jax.experimental.pallas import tpu as pltpu
from jax.experimental.pallas import tpu_sc as plsc

SEQ = 200
D = 64
LANES = 16
NC, NS = 2, 16
NW = NC * NS
BB = 128
NG = D // 8


def _body(tok_hbm, word_hbm, pos_hbm, out_hbm,
          tokblk, idx_t, pos_v, gb0, gb1, ch0, ch1, sg0, sg1, sw0, sw1):
    gb = [gb0, gb1]
    ch = [ch0, ch1]
    sg = [sg0, sg1]
    sw = [sw0, sw1]
    wid = lax.axis_index("s") * NC + lax.axis_index("c")

    pltpu.sync_copy(tok_hbm.at[pl.ds(wid * BB, BB)], tokblk)
    pltpu.sync_copy(pos_hbm, pos_v)

    lane = lax.iota(jnp.int32, LANES)

    def tpose(b, c):
        for k in range(13):
            off = k * LANES if k < 12 else SEQ - LANES
            val = tokblk[b, pl.ds(off, LANES)]
            rowi = off + lane
            coli = jnp.full((LANES,), b, jnp.int32)
            if k == 12:
                plsc.store_scatter(idx_t, [rowi, coli], val,
                                   mask=rowi >= k * LANES)
            else:
                plsc.store_scatter(idx_t, [rowi, coli], val)
        return c

    lax.fori_loop(0, BB, tpose, 0)

    dummy = word_hbm.at[pl.ds(0, BB)]

    def issue_gather(s, par):
        for q in range(4):
            sl = pl.ds(q * (BB // 4), BB // 4)
            pltpu.async_copy(word_hbm.at[idx_t.at[s, sl]], gb[par].at[sl],
                             sg[par])

    issue_gather(0, 0)

    gvec = [2 * k + lane // 8 for k in range(4)]
    cvec = lane % 8

    def step(s, par):
        @pl.when(s + 1 < SEQ)
        def _():
            issue_gather(s + 1, 1 - par)

        pltpu.make_async_copy(dummy, gb[par], sg[par]).wait()

        @pl.when(s >= 2)
        def _():
            for g in range(NG):
                pltpu.make_async_copy(ch[par].at[g, :, pl.ds(0, BB)],
                                      out_hbm.at[0, g, 0], sw[par]).wait()

        pos_r = [pos_v[s, pl.ds(k * LANES, LANES)] for k in range(4)]

        def add_tp(b2, c):
            for u in range(2):
                b = 2 * b2 + u
                bvec = jnp.full((LANES,), b, jnp.int32)
                for k in range(4):
                    val = gb[par][b, pl.ds(k * LANES, LANES)] + pos_r[k]
                    plsc.store_scatter(ch[par], [gvec[k], cvec, bvec], val)
            return c

        lax.fori_loop(0, BB // 2, add_tp, 0)
        for g in range(NG):
            pltpu.async_copy(ch[par].at[g, :, pl.ds(0, BB)],
                             out_hbm.at[s, g, wid], sw[par])

    def pair(p, carry):
        step(2 * p, 0)
        step(2 * p + 1, 1)
        return carry

    lax.fori_loop(0, SEQ // 2, pair, 0)
    for par in range(2):
        for g in range(NG):
            pltpu.make_async_copy(ch[par].at[g, :, pl.ds(0, BB)],
                                  out_hbm.at[0, g, 0], sw[par]).wait()


def kernel(tokens, word_table, pos_table):
    b, s = tokens.shape
    d = word_table.shape[1]
    tok = tokens.astype(jnp.int32)

    mesh = plsc.VectorSubcoreMesh(core_axis_name="c", subcore_axis_name="s",
                                  num_cores=NC, num_subcores=NS)
    run = pl.kernel(
        _body,
        out_type=jax.ShapeDtypeStruct((s, NG, b // BB, 8, BB), jnp.float32),
        mesh=mesh,
        scratch_types=(
            [pltpu.VMEM((BB, s), jnp.int32),
             pltpu.VMEM((s, BB), jnp.int32),
             pltpu.VMEM((SEQ, d), jnp.float32),
             pltpu.VMEM((BB, d), jnp.float32),
             pltpu.VMEM((BB, d), jnp.float32),
             pltpu.VMEM((NG, 8, BB + 1), jnp.float32),
             pltpu.VMEM((NG, 8, BB + 1), jnp.float32)]
            + [pltpu.SemaphoreType.DMA for _ in range(4)]
        ),
        compiler_params=pltpu.CompilerParams(use_tc_tiling_on_sc=False,
                                             needs_layout_passes=False),
    )
    out = run(tok, word_table, pos_table)
    return out.transpose(2, 4, 0, 1, 3).reshape(b, s, d)


# speedup vs baseline: 2.1655x; 1.4171x over previous
import jax
import jax.numpy as jnp
from jax import lax
from jax.experimental import pallas as pl
from jax.experimental.pallas import tpu as pltpu
from jax.experimental.pallas import tpu_sc as plsc

SEQ = 200
D = 64
LANES = 16
NC, NS = 2, 16
NW = NC * NS
BB = 128
NG = D // 8


def _body(tok_hbm, word_hbm, pos_hbm, out_hbm,
          tokblk, idx_t, pos_v, gb0, gb1, ch0, ch1, sg0, sg1, sw0, sw1):
    gb = [gb0, gb1]
    ch = [ch0, ch1]
    sg = [sg0, sg1]
    sw = [sw0, sw1]
    wid = lax.axis_index("s") * NC + lax.axis_index("c")

    pltpu.sync_copy(tok_hbm.at[pl.ds(wid * BB, BB)], tokblk)
    pltpu.sync_copy(pos_hbm, pos_v)

    lane = lax.iota(jnp.int32, LANES)

    @plsc.parallel_loop(0, BB, 1, unroll=2)
    def _(b):
        for k in range(13):
            off = k * LANES if k < 12 else SEQ - LANES
            val = tokblk[b, pl.ds(off, LANES)]
            rowi = off + lane
            coli = jnp.full((LANES,), b, jnp.int32)
            if k == 12:
                plsc.store_scatter(idx_t, [rowi, coli], val,
                                   mask=rowi >= k * LANES)
            else:
                plsc.store_scatter(idx_t, [rowi, coli], val)

    dummy = word_hbm.at[pl.ds(0, BB)]

    def issue_gather(s, par):
        for q in range(4):
            sl = pl.ds(q * (BB // 4), BB // 4)
            pltpu.async_copy(word_hbm.at[idx_t.at[s, sl]], gb[par].at[sl],
                             sg[par])

    issue_gather(0, 0)

    gvec = [2 * k + lane // 8 for k in range(4)]
    cvec = lane % 8

    def step(s, par):
        @pl.when(s + 1 < SEQ)
        def _():
            issue_gather(s + 1, 1 - par)

        pltpu.make_async_copy(dummy, gb[par], sg[par]).wait()

        @pl.when(s >= 2)
        def _():
            for g in range(NG):
                pltpu.make_async_copy(ch[par].at[g, :, pl.ds(0, BB)],
                                      out_hbm.at[0, g, 0], sw[par]).wait()

        pos_r = [pos_v[s, pl.ds(k * LANES, LANES)] for k in range(4)]

        @plsc.parallel_loop(0, BB, 1, unroll=4)
        def _(b):
            bvec = jnp.full((LANES,), b, jnp.int32)
            for k in range(4):
                val = gb[par][b, pl.ds(k * LANES, LANES)] + pos_r[k]
                plsc.store_scatter(ch[par], [gvec[k], cvec, bvec], val)
        for g in range(NG):
            pltpu.async_copy(ch[par].at[g, :, pl.ds(0, BB)],
                             out_hbm.at[s, g, wid], sw[par])

    def pair(p, carry):
        step(2 * p, 0)
        step(2 * p + 1, 1)
        return carry

    lax.fori_loop(0, SEQ // 2, pair, 0)
    for par in range(2):
        for g in range(NG):
            pltpu.make_async_copy(ch[par].at[g, :, pl.ds(0, BB)],
                                  out_hbm.at[0, g, 0], sw[par]).wait()


def kernel(tokens, word_table, pos_table):
    b, s = tokens.shape
    d = word_table.shape[1]
    tok = tokens.astype(jnp.int32)

    mesh = plsc.VectorSubcoreMesh(core_axis_name="c", subcore_axis_name="s",
                                  num_cores=NC, num_subcores=NS)
    run = pl.kernel(
        _body,
        out_type=jax.ShapeDtypeStruct((s, NG, b // BB, 8, BB), jnp.float32),
        mesh=mesh,
        scratch_types=(
            [pltpu.VMEM((BB, s), jnp.int32),
             pltpu.VMEM((s, BB), jnp.int32),
             pltpu.VMEM((SEQ, d), jnp.float32),
             pltpu.VMEM((BB, d), jnp.float32),
             pltpu.VMEM((BB, d), jnp.float32),
             pltpu.VMEM((NG, 8, BB + 1), jnp.float32),
             pltpu.VMEM((NG, 8, BB + 1), jnp.float32)]
            + [pltpu.SemaphoreType.DMA for _ in range(4)]
        ),
        compiler_params=pltpu.CompilerParams(use_tc_tiling_on_sc=False,
                                             needs_layout_passes=False),
    )
    out = run(tok, word_table, pos_table)
    return out.transpose(2, 4, 0, 1, 3).reshape(b, s, d)


# speedup vs baseline: 2.1748x; 1.0043x over previous
import jax
import jax.numpy as jnp
from jax import lax
from jax.experimental import pallas as pl
from jax.experimental.pallas import tpu as pltpu
from jax.experimental.pallas import tpu_sc as plsc

SEQ = 200
D = 64
LANES = 16
NC, NS = 2, 16
NW = NC * NS
BB = 128
NG = D // 8


def _body(tok_hbm, word_hbm, pos_hbm, out_hbm,
          tokblk, idx_t, pos_v, gb0, gb1, ch0, ch1, sg0, sg1, sw0, sw1):
    gb = [gb0, gb1]
    ch = [ch0, ch1]
    sg = [sg0, sg1]
    sw = [sw0, sw1]
    wid = lax.axis_index("s") * NC + lax.axis_index("c")

    pltpu.sync_copy(tok_hbm.at[pl.ds(wid * BB, BB)], tokblk)
    pltpu.sync_copy(pos_hbm, pos_v)

    lane = lax.iota(jnp.int32, LANES)

    @plsc.parallel_loop(0, BB, 1, unroll=2)
    def _(b):
        for k in range(13):
            off = k * LANES if k < 12 else SEQ - LANES
            val = tokblk[b, pl.ds(off, LANES)]
            rowi = off + lane
            coli = jnp.full((LANES,), b, jnp.int32)
            if k == 12:
                plsc.store_scatter(idx_t, [rowi, coli], val,
                                   mask=rowi >= k * LANES)
            else:
                plsc.store_scatter(idx_t, [rowi, coli], val)

    dummy = word_hbm.at[pl.ds(0, BB)]

    def issue_gather(s, par):
        for q in range(4):
            sl = pl.ds(q * (BB // 4), BB // 4)
            pltpu.async_copy(word_hbm.at[idx_t.at[s, sl]], gb[par].at[sl],
                             sg[par])

    issue_gather(0, 0)

    gvec = [2 * k + lane // 8 for k in range(4)]
    cvec = lane % 8

    def step(s, par):
        @pl.when(s + 1 < SEQ)
        def _():
            issue_gather(s + 1, 1 - par)

        pltpu.make_async_copy(dummy, gb[par], sg[par]).wait()

        @pl.when(s >= 2)
        def _():
            pltpu.make_async_copy(ch[par].at[:, :, pl.ds(0, BB)],
                                  out_hbm.at[0, :, 0], sw[par]).wait()

        pos_r = [pos_v[s, pl.ds(k * LANES, LANES)] for k in range(4)]

        @plsc.parallel_loop(0, BB, 1, unroll=4)
        def _(b):
            bvec = jnp.full((LANES,), b, jnp.int32)
            for k in range(4):
                val = gb[par][b, pl.ds(k * LANES, LANES)] + pos_r[k]
                plsc.store_scatter(ch[par], [gvec[k], cvec, bvec], val)
        pltpu.async_copy(ch[par].at[:, :, pl.ds(0, BB)],
                         out_hbm.at[s, :, wid], sw[par])

    def pair(p, carry):
        step(2 * p, 0)
        step(2 * p + 1, 1)
        return carry

    lax.fori_loop(0, SEQ // 2, pair, 0)
    for par in range(2):
        pltpu.make_async_copy(ch[par].at[:, :, pl.ds(0, BB)],
                              out_hbm.at[0, :, 0], sw[par]).wait()


def kernel(tokens, word_table, pos_table):
    b, s = tokens.shape
    d = word_table.shape[1]
    tok = tokens.astype(jnp.int32)

    mesh = plsc.VectorSubcoreMesh(core_axis_name="c", subcore_axis_name="s",
                                  num_cores=NC, num_subcores=NS)
    run = pl.kernel(
        _body,
        out_type=jax.ShapeDtypeStruct((s, NG, b // BB, 8, BB), jnp.float32),
        mesh=mesh,
        scratch_types=(
            [pltpu.VMEM((BB, s), jnp.int32),
             pltpu.VMEM((s, BB), jnp.int32),
             pltpu.VMEM((SEQ, d), jnp.float32),
             pltpu.VMEM((BB, d), jnp.float32),
             pltpu.VMEM((BB, d), jnp.float32),
             pltpu.VMEM((NG, 8, BB + 1), jnp.float32),
             pltpu.VMEM((NG, 8, BB + 1), jnp.float32)]
            + [pltpu.SemaphoreType.DMA for _ in range(4)]
        ),
        compiler_params=pltpu.CompilerParams(use_tc_tiling_on_sc=False,
                                             needs_layout_passes=False),
    )
    out = run(tok, word_table, pos_table)
    return out.transpose(2, 4, 0, 1, 3).reshape(b, s, d)
